# baseline (speedup 1.0000x reference)
import functools

import jax
import jax.numpy as jnp
from jax import lax
from jax.experimental import pallas as pl
from jax.experimental.pallas import tpu as pltpu
from jax.experimental.pallas import tpu_sc as plsc

D_MODEL = 768

_info = plsc.get_sparse_core_info()
NC, NS = _info.num_cores, _info.num_subcores
NW = NC * NS

CHUNK = 32
NBUF = 4


def kernel(tokens, W_E):
    bsz, seq = tokens.shape
    n_tokens = bsz * seq
    b_per_w = n_tokens // NW
    w_per_row = seq // b_per_w
    n_chunks = b_per_w // CHUNK
    mesh = plsc.VectorSubcoreMesh(core_axis_name="c", subcore_axis_name="s")

    @functools.partial(
        pl.kernel,
        out_type=jax.ShapeDtypeStruct((bsz, seq, D_MODEL), jnp.float32),
        mesh=mesh,
        scratch_types=[
            pltpu.VMEM((b_per_w,), jnp.int32),
            [pltpu.VMEM((CHUNK, D_MODEL), jnp.float32) for _ in range(NBUF)],
            [pltpu.SemaphoreType.DMA for _ in range(NBUF)],
            [pltpu.SemaphoreType.DMA for _ in range(NBUF)],
        ],
    )
    def k(idx_hbm, table_hbm, out_hbm, idx_v, bufs, gsems, ssems):
        wid = lax.axis_index("s") * NC + lax.axis_index("c")
        row = wid // w_per_row
        soff = (wid % w_per_row) * b_per_w
        pltpu.sync_copy(idx_hbm.at[row, pl.ds(soff, b_per_w)], idx_v)
        gathers = [None] * n_chunks
        scatters = [None] * n_chunks
        for c in range(min(NBUF, n_chunks)):
            gathers[c] = pltpu.async_copy(
                table_hbm.at[idx_v.at[pl.ds(c * CHUNK, CHUNK)]], bufs[c], gsems[c]
            )
        for c in range(n_chunks):
            b = c % NBUF
            gathers[c].wait()
            scatters[c] = pltpu.async_copy(
                bufs[b],
                out_hbm.at[row, pl.ds(soff + c * CHUNK, CHUNK)],
                ssems[b],
            )
            nxt = c + NBUF
            if nxt < n_chunks:
                scatters[c].wait()
                gathers[nxt] = pltpu.async_copy(
                    table_hbm.at[idx_v.at[pl.ds(nxt * CHUNK, CHUNK)]],
                    bufs[b],
                    gsems[b],
                )
        for c in range(max(0, n_chunks - NBUF), n_chunks):
            scatters[c].wait()

    return k(tokens, W_E)


# speedup vs baseline: 1.0131x; 1.0131x over previous
import functools

import jax
import jax.numpy as jnp
from jax import lax
from jax.experimental import pallas as pl
from jax.experimental.pallas import tpu as pltpu
from jax.experimental.pallas import tpu_sc as plsc

D_MODEL = 768

_info = plsc.get_sparse_core_info()
NC, NS = _info.num_cores, _info.num_subcores
NW = NC * NS

CHUNK = 32
NBUF = 4


def kernel(tokens, W_E):
    bsz, seq = tokens.shape
    n_tokens = bsz * seq
    b_per_w = n_tokens // NW
    w_per_row = seq // b_per_w
    n_chunks = b_per_w // CHUNK
    mesh = plsc.VectorSubcoreMesh(core_axis_name="c", subcore_axis_name="s")

    @functools.partial(
        pl.kernel,
        out_type=jax.ShapeDtypeStruct((bsz, seq, D_MODEL), jnp.float32),
        mesh=mesh,
        scratch_types=[
            pltpu.VMEM((b_per_w,), jnp.int32),
            [pltpu.VMEM((CHUNK, D_MODEL), jnp.float32) for _ in range(NBUF)],
            [pltpu.SemaphoreType.DMA for _ in range(NBUF)],
            [pltpu.SemaphoreType.DMA for _ in range(NBUF)],
        ],
    )
    def k(idx_hbm, table_hbm, out_hbm, idx_v, bufs, gsems, ssems):
        wid = lax.axis_index("s") * NC + lax.axis_index("c")
        row = wid // w_per_row
        soff = (wid % w_per_row) * b_per_w
        pltpu.sync_copy(idx_hbm.at[row, pl.ds(soff, b_per_w)], idx_v)

        def gather(c, b):
            return pltpu.async_copy(
                table_hbm.at[idx_v.at[pl.ds(c * CHUNK, CHUNK)]], bufs[b], gsems[b]
            )

        def gather_wait(b):
            pltpu.make_async_copy(
                table_hbm.at[idx_v.at[pl.ds(0, CHUNK)]], bufs[b], gsems[b]
            ).wait()

        def scatter(c, b):
            return pltpu.async_copy(
                bufs[b], out_hbm.at[row, pl.ds(soff + c * CHUNK, CHUNK)], ssems[b]
            )

        for b in range(NBUF):
            gather(b, b)

        @pl.loop(0, n_chunks - NBUF, step=NBUF)
        def _(g):
            for b in range(NBUF):
                c = g + b
                gather_wait(b)
                scatter(c, b).wait()
                gather(c + NBUF, b)

        last = []
        for b in range(NBUF):
            c = n_chunks - NBUF + b
            gather_wait(b)
            last.append(scatter(c, b))
        for h in last:
            h.wait()

    return k(tokens, W_E)
